# baseline (speedup 1.0000x reference)
import functools

import jax
import jax.numpy as jnp
from jax import lax
from jax.experimental import pallas as pl
from jax.experimental.pallas import tpu as pltpu
from jax.experimental.pallas import tpu_sc as plsc

N = 10000
E = 320000
D_IN = 128
D_H = 256
DH = 128

NC = 2
NS = 16
NW = NC * NS
CH = 80
CHD = 100
RPT = E // CH // NS
GD = 50
GF = 50
NGD = E // CHD // NW // GD
NGF = RPT // GF
NPAD = 10240
BM = 400
NB = N // BM


def _mesh():
    return plsc.VectorSubcoreMesh(
        core_axis_name="c", subcore_axis_name="s", num_cores=NC, num_subcores=NS
    )


def _sc_deg_body(keepf_hbm, src_hbm, dst_hbm, zeros_hbm, out0_hbm, out1_hbm,
                 src_v, dst_v, vals0_v, vals1_v, acc_sh, sem0, sem1):
    c = lax.axis_index("c")
    s = lax.axis_index("s")
    w = s * NC + c
    npt = NPAD // NS
    t0 = s * npt
    pltpu.sync_copy(zeros_hbm.at[pl.ds(t0, npt)], acc_sh.at[pl.ds(t0, npt)])
    plsc.subcore_barrier()

    vals = (vals0_v, vals1_v)
    sems = (sem0, sem1)

    def group(g, carry):
        pltpu.sync_copy(src_hbm.at[w, g], src_v)
        pltpu.sync_copy(dst_hbm.at[w, g], dst_v)
        pltpu.async_copy(keepf_hbm.at[src_v.at[0]], vals0_v, sem0)

        def body(jj, inner):
            for b in range(2):
                j = 2 * jj + b
                pltpu.make_async_copy(
                    keepf_hbm.at[src_v.at[j]], vals[b], sems[b]).wait()

                @pl.when(j + 1 < GD)
                def _():
                    pltpu.async_copy(
                        keepf_hbm.at[src_v.at[j + 1]], vals[1 - b], sems[1 - b])

                pltpu.sync_copy(vals[b], acc_sh.at[dst_v.at[j]], add=True)
            return inner

        return lax.fori_loop(0, GD // 2, body, carry)

    lax.fori_loop(0, NGD, group, 0)
    plsc.subcore_barrier()

    @pl.when(c == 0)
    def _():
        pltpu.sync_copy(acc_sh.at[pl.ds(t0, npt)], out0_hbm.at[pl.ds(t0, npt)])

    @pl.when(c == 1)
    def _():
        pltpu.sync_copy(acc_sh.at[pl.ds(t0, npt)], out1_hbm.at[pl.ds(t0, npt)])


@jax.jit
def _sc_deg(keepf, src3d, dst3d, zeros_deg):
    return pl.kernel(
        _sc_deg_body,
        out_type=(
            jax.ShapeDtypeStruct((NPAD,), jnp.float32),
            jax.ShapeDtypeStruct((NPAD,), jnp.float32),
        ),
        mesh=_mesh(),
        scratch_types=[
            pltpu.VMEM((GD, CHD), jnp.int32),
            pltpu.VMEM((GD, CHD), jnp.int32),
            pltpu.VMEM((CHD,), jnp.float32),
            pltpu.VMEM((CHD,), jnp.float32),
            pltpu.VMEM_SHARED((NPAD,), jnp.float32),
            pltpu.SemaphoreType.DMA,
            pltpu.SemaphoreType.DMA,
        ],
    )(keepf, src3d, dst3d, zeros_deg)


def _sc_agg_feat_body(xs0_hbm, xs1_hbm, src_hbm, dst_hbm, zeros_hbm,
                      out0_hbm, out1_hbm,
                      src_v, dst_v, rows0_v, rows1_v, acc_sh, sem0, sem1):
    c = lax.axis_index("c")
    s = lax.axis_index("s")
    nr = NPAD // NS
    r0 = s * nr
    pltpu.sync_copy(zeros_hbm.at[pl.ds(r0, nr)], acc_sh.at[pl.ds(r0, nr)])
    plsc.subcore_barrier()

    rows = (rows0_v, rows1_v)
    sems = (sem0, sem1)

    def run(xs_hbm):
        def group(g, carry):
            pltpu.sync_copy(src_hbm.at[s, g], src_v)
            pltpu.sync_copy(dst_hbm.at[s, g], dst_v)
            pltpu.async_copy(xs_hbm.at[src_v.at[0]], rows0_v, sem0)

            def body(jj, inner):
                for b in range(2):
                    j = 2 * jj + b
                    pltpu.make_async_copy(
                        xs_hbm.at[src_v.at[j]], rows[b], sems[b]).wait()

                    @pl.when(j + 1 < GF)
                    def _():
                        pltpu.async_copy(
                            xs_hbm.at[src_v.at[j + 1]], rows[1 - b],
                            sems[1 - b])

                    pltpu.sync_copy(rows[b], acc_sh.at[dst_v.at[j]], add=True)
                return inner

            return lax.fori_loop(0, GF // 2, body, carry)

        lax.fori_loop(0, NGF, group, 0)

    @pl.when(c == 0)
    def _():
        run(xs0_hbm)

    @pl.when(c == 1)
    def _():
        run(xs1_hbm)

    plsc.subcore_barrier()

    @pl.when(c == 0)
    def _():
        pltpu.sync_copy(acc_sh.at[pl.ds(r0, nr)], out0_hbm.at[pl.ds(r0, nr)])

    @pl.when(c == 1)
    def _():
        pltpu.sync_copy(acc_sh.at[pl.ds(r0, nr)], out1_hbm.at[pl.ds(r0, nr)])


@jax.jit
def _sc_agg_feat(xs0, xs1, srcS, dstS, zeros2d):
    return pl.kernel(
        _sc_agg_feat_body,
        out_type=(
            jax.ShapeDtypeStruct((NPAD, DH), jnp.float32),
            jax.ShapeDtypeStruct((NPAD, DH), jnp.float32),
        ),
        mesh=_mesh(),
        scratch_types=[
            pltpu.VMEM((GF, CH), jnp.int32),
            pltpu.VMEM((GF, CH), jnp.int32),
            pltpu.VMEM((CH, DH), jnp.float32),
            pltpu.VMEM((CH, DH), jnp.float32),
            pltpu.VMEM_SHARED((NPAD, DH), jnp.float32),
            pltpu.SemaphoreType.DMA,
            pltpu.SemaphoreType.DMA,
        ],
    )(xs0, xs1, srcS, dstS, zeros2d)


def _prep_split_body(deg0_ref, deg1_ref, x_ref, dis_ref, xs0_ref, xs1_ref):
    deg = deg0_ref[:N] + deg1_ref[:N] + 1.0
    dis = lax.rsqrt(deg)
    dis_ref[...] = dis
    xs = x_ref[...] * dis[:, None]
    xs0_ref[...] = xs[:, :DH]
    xs1_ref[...] = xs[:, DH:]


@jax.jit
def _tc_prep_split(deg0, deg1, x_cur):
    return pl.pallas_call(
        _prep_split_body,
        out_shape=(
            jax.ShapeDtypeStruct((N,), jnp.float32),
            jax.ShapeDtypeStruct((N, DH), jnp.float32),
            jax.ShapeDtypeStruct((N, DH), jnp.float32),
        ),
    )(deg0, deg1, x_cur)


def _mm_cat_body(a0_ref, a1_ref, xs0_ref, xs1_ref, dis_ref, keepf_ref,
                 W_ref, b_ref, p_ref, h_ref, score_ref):
    pre = jnp.concatenate(
        [a0_ref[...] + xs0_ref[...], a1_ref[...] + xs1_ref[...]], axis=1)
    d = dis_ref[0, 0, :]
    pre = pre * d[:, None]
    h = jnp.dot(pre, W_ref[...], preferred_element_type=jnp.float32,
                precision=lax.Precision.HIGHEST)
    h = jnp.maximum(h + b_ref[...][None, :], 0.0)
    h_ref[...] = h
    p = p_ref[...]
    pden = jnp.sqrt(jnp.sum(p * p)) + 1e-16
    sc = jnp.sum(h * p[None, :], axis=1) / pden
    kf = keepf_ref[0, 0, :]
    score_ref[0, 0, :] = jnp.where(kf > 0, sc, -jnp.inf)


@jax.jit
def _tc_matmul_cat(a0, a1, xs0, xs1, dis, keepf, W, b, p):
    blk = pl.BlockSpec((BM, DH), lambda i: (i, 0))
    nod = pl.BlockSpec((1, 1, BM), lambda i: (i, 0, 0))
    dis3 = dis.reshape(NB, 1, BM)
    keepf3 = keepf.reshape(NB, 1, BM)
    h, score3 = pl.pallas_call(
        _mm_cat_body,
        grid=(NB,),
        in_specs=[blk, blk, blk, blk, nod, nod,
                  pl.BlockSpec((D_H, D_H), lambda i: (0, 0)),
                  pl.BlockSpec((D_H,), lambda i: (0,)),
                  pl.BlockSpec((D_H,), lambda i: (0,))],
        out_specs=[pl.BlockSpec((BM, D_H), lambda i: (i, 0)), nod],
        out_shape=[
            jax.ShapeDtypeStruct((N, D_H), jnp.float32),
            jax.ShapeDtypeStruct((NB, 1, BM), jnp.float32),
        ],
    )(a0, a1, xs0, xs1, dis3, keepf3, W, b, p)
    return h, score3.reshape(N)


def _pool_body(h_ref, score_ref, gw_ref, xn_ref, keepf_ref, out_ref, *, k):
    score = score_ref[...]
    bits = lax.bitcast_convert_type(score, jnp.int32)
    skey = jnp.where(bits < 0, bits ^ jnp.int32(0x7FFFFFFF), bits)

    c_nonneg = jnp.sum((skey >= 0).astype(jnp.int32))
    t0 = jnp.where(c_nonneg >= k, jnp.int32(0), jnp.int32(-2147483648))

    def bit_step(bb, t):
        cand = t | (jnp.int32(1) << (30 - bb))
        cnt = jnp.sum((skey >= cand).astype(jnp.int32))
        return jnp.where(cnt >= k, cand, t)

    thr = lax.fori_loop(0, 31, bit_step, t0)
    c_gt = jnp.sum((skey > thr).astype(jnp.int32))
    t_need = k - c_gt
    iota = lax.iota(jnp.int32, N)
    is_tie = skey == thr

    def idx_step(bb, m):
        cand = m | (jnp.int32(1) << (13 - bb))
        f = jnp.sum((is_tie & (iota < cand)).astype(jnp.int32))
        return jnp.where(f < t_need, cand, m)

    m = lax.fori_loop(0, 14, idx_step, jnp.int32(0))
    keep = (skey > thr) | (is_tie & (iota <= m) & (t_need > 0))
    keepf = keep.astype(jnp.float32)
    keepf_ref[...] = keepf

    th = jnp.tanh(score)
    xn = h_ref[...] * (th * keepf)[:, None]
    xn_ref[...] = xn
    gate = jnp.sum(xn * gw_ref[...][None, :], axis=1)
    gate = jnp.where(keep, gate, -jnp.inf)
    mx = jnp.max(gate)
    al = jnp.exp(gate - mx) * keepf
    z = jnp.sum(al)
    out_ref[...] = (jnp.sum(xn * al[:, None], axis=0) / z)[None, :]


@functools.partial(jax.jit, static_argnames=("k",))
def _tc_pool(h, score, gw, k):
    return pl.pallas_call(
        functools.partial(_pool_body, k=k),
        out_shape=(
            jax.ShapeDtypeStruct((N, D_H), jnp.float32),
            jax.ShapeDtypeStruct((N,), jnp.float32),
            jax.ShapeDtypeStruct((1, D_H), jnp.float32),
        ),
    )(h, score, gw)


def kernel(x, edge_index, batch, W_in, b_in, p_in, W_h1, b_h1, p_h1,
           W_h2, b_h2, p_h2, gate_w, gate_b):
    src3d = edge_index[0].reshape(NW, NGD, GD, CHD)
    dst3d = edge_index[1].reshape(NW, NGD, GD, CHD)
    srcS = edge_index[0].reshape(NS, NGF, GF, CH)
    dstS = edge_index[1].reshape(NS, NGF, GF, CH)
    zeros_deg = jnp.zeros((NPAD,), jnp.float32)
    z2d = jnp.zeros((NPAD, DH), jnp.float32)
    keepf = jnp.ones((N,), jnp.float32)
    gw = gate_w[:, 0]

    x_cur = jnp.concatenate([x, jnp.zeros((N, D_H - D_IN), jnp.float32)], axis=1)
    W1 = jnp.concatenate([W_in, jnp.zeros((D_H - D_IN, D_H), jnp.float32)], axis=0)

    out = None
    for (W, b, p, k) in ((W1, b_in, p_in, 8000),
                         (W_h1, b_h1, p_h1, 6400),
                         (W_h2, b_h2, p_h2, 5120)):
        deg0, deg1 = _sc_deg(keepf, src3d, dst3d, zeros_deg)
        dis, xs0, xs1 = _tc_prep_split(deg0, deg1, x_cur)
        a0, a1 = _sc_agg_feat(xs0, xs1, srcS, dstS, z2d)
        h, score = _tc_matmul_cat(a0[:N], a1[:N], xs0, xs1, dis, keepf, W, b, p)
        x_cur, keepf, out_l = _tc_pool(h, score, gw, k)
        out = out_l if out is None else out + out_l
    return out


# speedup vs baseline: 1.0426x; 1.0426x over previous
import functools

import jax
import jax.numpy as jnp
from jax import lax
from jax.experimental import pallas as pl
from jax.experimental.pallas import tpu as pltpu
from jax.experimental.pallas import tpu_sc as plsc

N = 10000
E = 320000
D_IN = 128
D_H = 256
DH = 128

NC = 2
NS = 16
NW = NC * NS
CH = 100
GD = 50
GF = 50
NGD = E // CH // NW // GD
NGF = E // CH // NS // GF
NPAD = 10240
BM = 400
NB = N // BM


def _mesh():
    return plsc.VectorSubcoreMesh(
        core_axis_name="c", subcore_axis_name="s", num_cores=NC, num_subcores=NS
    )


def _sc_deg_body(keepf_hbm, src_hbm, dst_hbm, zeros_hbm, out0_hbm, out1_hbm,
                 src_v, dst_v, vals0_v, vals1_v, acc_sh,
                 gsem0, gsem1, ssem0, ssem1):
    c = lax.axis_index("c")
    s = lax.axis_index("s")
    w = s * NC + c
    npt = NPAD // NS
    t0 = s * npt
    pltpu.sync_copy(zeros_hbm.at[pl.ds(t0, npt)], acc_sh.at[pl.ds(t0, npt)])
    plsc.subcore_barrier()

    vals = (vals0_v, vals1_v)
    gs = (gsem0, gsem1)
    ss = (ssem0, ssem1)

    def swait(b, j):
        pltpu.make_async_copy(
            vals[b], acc_sh.at[dst_v.at[j]], ss[b]).wait()

    def group(g, carry):
        @pl.when(g > 0)
        def _():
            swait(0, 0)
            swait(1, 1)
        pltpu.sync_copy(src_hbm.at[w, g], src_v)
        pltpu.sync_copy(dst_hbm.at[w, g], dst_v)
        pltpu.async_copy(keepf_hbm.at[src_v.at[0]], vals0_v, gsem0)

        def body(jj, inner):
            for b in range(2):
                j = 2 * jj + b
                pltpu.make_async_copy(
                    keepf_hbm.at[src_v.at[j]], vals[b], gs[b]).wait()

                @pl.when(j + 1 < GD)
                def _():
                    @pl.when(j >= 1)
                    def _():
                        swait(1 - b, j)
                    pltpu.async_copy(
                        keepf_hbm.at[src_v.at[j + 1]], vals[1 - b], gs[1 - b])

                pltpu.async_copy(vals[b], acc_sh.at[dst_v.at[j]], ss[b],
                                 add=True)
            return inner

        return lax.fori_loop(0, GD // 2, body, carry)

    lax.fori_loop(0, NGD, group, 0)
    swait(0, 0)
    swait(1, 1)
    plsc.subcore_barrier()

    @pl.when(c == 0)
    def _():
        pltpu.sync_copy(acc_sh.at[pl.ds(t0, npt)], out0_hbm.at[pl.ds(t0, npt)])

    @pl.when(c == 1)
    def _():
        pltpu.sync_copy(acc_sh.at[pl.ds(t0, npt)], out1_hbm.at[pl.ds(t0, npt)])


@jax.jit
def _sc_deg(keepf, src3d, dst3d, zeros_deg):
    return pl.kernel(
        _sc_deg_body,
        out_type=(
            jax.ShapeDtypeStruct((NPAD,), jnp.float32),
            jax.ShapeDtypeStruct((NPAD,), jnp.float32),
        ),
        mesh=_mesh(),
        scratch_types=[
            pltpu.VMEM((GD, CH), jnp.int32),
            pltpu.VMEM((GD, CH), jnp.int32),
            pltpu.VMEM((CH,), jnp.float32),
            pltpu.VMEM((CH,), jnp.float32),
            pltpu.VMEM_SHARED((NPAD,), jnp.float32),
            pltpu.SemaphoreType.DMA,
            pltpu.SemaphoreType.DMA,
            pltpu.SemaphoreType.DMA,
            pltpu.SemaphoreType.DMA,
        ],
    )(keepf, src3d, dst3d, zeros_deg)


def _sc_agg_feat_body(xs0_hbm, xs1_hbm, src_hbm, dst_hbm, zeros_hbm,
                      out0_hbm, out1_hbm,
                      src_v, dst_v, rows0_v, rows1_v, acc_sh,
                      gsem0, gsem1, ssem0, ssem1):
    c = lax.axis_index("c")
    s = lax.axis_index("s")
    nr = NPAD // NS
    r0 = s * nr
    pltpu.sync_copy(zeros_hbm.at[pl.ds(r0, nr)], acc_sh.at[pl.ds(r0, nr)])
    plsc.subcore_barrier()

    rows = (rows0_v, rows1_v)
    gs = (gsem0, gsem1)
    ss = (ssem0, ssem1)

    def swait(b, j):
        pltpu.make_async_copy(
            rows[b], acc_sh.at[dst_v.at[j]], ss[b]).wait()

    def run(xs_hbm):
        def group(g, carry):
            @pl.when(g > 0)
            def _():
                swait(0, 0)
                swait(1, 1)
            pltpu.sync_copy(src_hbm.at[s, g], src_v)
            pltpu.sync_copy(dst_hbm.at[s, g], dst_v)
            pltpu.async_copy(xs_hbm.at[src_v.at[0]], rows0_v, gsem0)

            def body(jj, inner):
                for b in range(2):
                    j = 2 * jj + b
                    pltpu.make_async_copy(
                        xs_hbm.at[src_v.at[j]], rows[b], gs[b]).wait()

                    @pl.when(j + 1 < GF)
                    def _():
                        @pl.when(j >= 1)
                        def _():
                            swait(1 - b, j)
                        pltpu.async_copy(
                            xs_hbm.at[src_v.at[j + 1]], rows[1 - b],
                            gs[1 - b])

                    pltpu.async_copy(rows[b], acc_sh.at[dst_v.at[j]], ss[b],
                                     add=True)
                return inner

            return lax.fori_loop(0, GF // 2, body, carry)

        lax.fori_loop(0, NGF, group, 0)
        swait(0, 0)
        swait(1, 1)

    @pl.when(c == 0)
    def _():
        run(xs0_hbm)

    @pl.when(c == 1)
    def _():
        run(xs1_hbm)

    plsc.subcore_barrier()

    @pl.when(c == 0)
    def _():
        pltpu.sync_copy(acc_sh.at[pl.ds(r0, nr)], out0_hbm.at[pl.ds(r0, nr)])

    @pl.when(c == 1)
    def _():
        pltpu.sync_copy(acc_sh.at[pl.ds(r0, nr)], out1_hbm.at[pl.ds(r0, nr)])


@jax.jit
def _sc_agg_feat(xs0, xs1, srcS, dstS, zeros2d):
    return pl.kernel(
        _sc_agg_feat_body,
        out_type=(
            jax.ShapeDtypeStruct((NPAD, DH), jnp.float32),
            jax.ShapeDtypeStruct((NPAD, DH), jnp.float32),
        ),
        mesh=_mesh(),
        scratch_types=[
            pltpu.VMEM((GF, CH), jnp.int32),
            pltpu.VMEM((GF, CH), jnp.int32),
            pltpu.VMEM((CH, DH), jnp.float32),
            pltpu.VMEM((CH, DH), jnp.float32),
            pltpu.VMEM_SHARED((NPAD, DH), jnp.float32),
            pltpu.SemaphoreType.DMA,
            pltpu.SemaphoreType.DMA,
            pltpu.SemaphoreType.DMA,
            pltpu.SemaphoreType.DMA,
        ],
    )(xs0, xs1, srcS, dstS, zeros2d)


def _mm_pre_body(d0_ref, d1_ref, x_ref, W_ref, dis_ref, hw_ref,
                 hws0_ref, hws1_ref):
    deg = d0_ref[0, 0, :] + d1_ref[0, 0, :] + 1.0
    dis = lax.rsqrt(deg)
    dis_ref[0, 0, :] = dis
    hw = jnp.dot(x_ref[...], W_ref[...], preferred_element_type=jnp.float32)
    hw_ref[...] = hw
    hws = hw * dis[:, None]
    hws0_ref[...] = hws[:, :DH]
    hws1_ref[...] = hws[:, DH:]


@jax.jit
def _tc_mm_pre(deg0, deg1, x_cur, W):
    din = x_cur.shape[1]
    nod = pl.BlockSpec((1, 1, BM), lambda i: (i, 0, 0))
    half = pl.BlockSpec((BM, DH), lambda i: (i, 0))
    d03 = deg0[:N].reshape(NB, 1, BM)
    d13 = deg1[:N].reshape(NB, 1, BM)
    dis3, hw, hws0, hws1 = pl.pallas_call(
        _mm_pre_body,
        grid=(NB,),
        in_specs=[nod, nod,
                  pl.BlockSpec((BM, din), lambda i: (i, 0)),
                  pl.BlockSpec((din, D_H), lambda i: (0, 0))],
        out_specs=[nod, pl.BlockSpec((BM, D_H), lambda i: (i, 0)), half, half],
        out_shape=[
            jax.ShapeDtypeStruct((NB, 1, BM), jnp.float32),
            jax.ShapeDtypeStruct((N, D_H), jnp.float32),
            jax.ShapeDtypeStruct((N, DH), jnp.float32),
            jax.ShapeDtypeStruct((N, DH), jnp.float32),
        ],
    )(d03, d13, x_cur, W)
    return dis3, hw, hws0, hws1


def _finish_body(a0_ref, a1_ref, hw_ref, dis_ref, b_ref, h_ref):
    d = dis_ref[0, 0, :]
    agg = jnp.concatenate([a0_ref[...], a1_ref[...]], axis=1)
    h = agg * d[:, None] + hw_ref[...] * (d * d)[:, None] + b_ref[...][None, :]
    h_ref[...] = jnp.maximum(h, 0.0)


@jax.jit
def _tc_finish(a0, a1, hw, dis3, b):
    blk = pl.BlockSpec((BM, DH), lambda i: (i, 0))
    nod = pl.BlockSpec((1, 1, BM), lambda i: (i, 0, 0))
    return pl.pallas_call(
        _finish_body,
        grid=(NB,),
        in_specs=[blk, blk,
                  pl.BlockSpec((BM, D_H), lambda i: (i, 0)),
                  nod,
                  pl.BlockSpec((D_H,), lambda i: (0,))],
        out_specs=pl.BlockSpec((BM, D_H), lambda i: (i, 0)),
        out_shape=jax.ShapeDtypeStruct((N, D_H), jnp.float32),
    )(a0, a1, hw, dis3, b)


def _pool_body(h_ref, score_ref, gw_ref, xn_ref, keepf_ref, out_ref, *, k):
    score = score_ref[...]
    bits = lax.bitcast_convert_type(score, jnp.int32)
    skey = jnp.where(bits < 0, bits ^ jnp.int32(0x7FFFFFFF), bits)

    c_nonneg = jnp.sum((skey >= 0).astype(jnp.int32))
    t0 = jnp.where(c_nonneg >= k, jnp.int32(0), jnp.int32(-2147483648))

    def bit_step(bb, t):
        cand = t | (jnp.int32(1) << (30 - bb))
        cnt = jnp.sum((skey >= cand).astype(jnp.int32))
        return jnp.where(cnt >= k, cand, t)

    thr = lax.fori_loop(0, 31, bit_step, t0)
    c_gt = jnp.sum((skey > thr).astype(jnp.int32))
    t_need = k - c_gt
    iota = lax.iota(jnp.int32, N)
    is_tie = skey == thr

    def idx_step(bb, m):
        cand = m | (jnp.int32(1) << (13 - bb))
        f = jnp.sum((is_tie & (iota < cand)).astype(jnp.int32))
        return jnp.where(f < t_need, cand, m)

    m = lax.fori_loop(0, 14, idx_step, jnp.int32(0))
    keep = (skey > thr) | (is_tie & (iota <= m) & (t_need > 0))
    keepf = keep.astype(jnp.float32)
    keepf_ref[...] = keepf

    th = jnp.tanh(score)
    xn = h_ref[...] * (th * keepf)[:, None]
    xn_ref[...] = xn
    gate = jnp.sum(xn * gw_ref[...][None, :], axis=1)
    gate = jnp.where(keep, gate, -jnp.inf)
    mx = jnp.max(gate)
    al = jnp.exp(gate - mx) * keepf
    z = jnp.sum(al)
    out_ref[...] = (jnp.sum(xn * al[:, None], axis=0) / z)[None, :]


@functools.partial(jax.jit, static_argnames=("k",))
def _tc_pool(h, score, gw, k):
    return pl.pallas_call(
        functools.partial(_pool_body, k=k),
        out_shape=(
            jax.ShapeDtypeStruct((N, D_H), jnp.float32),
            jax.ShapeDtypeStruct((N,), jnp.float32),
            jax.ShapeDtypeStruct((1, D_H), jnp.float32),
        ),
    )(h, score, gw)


def kernel(x, edge_index, batch, W_in, b_in, p_in, W_h1, b_h1, p_h1,
           W_h2, b_h2, p_h2, gate_w, gate_b):
    src3d = edge_index[0].reshape(NW, NGD, GD, CH)
    dst3d = edge_index[1].reshape(NW, NGD, GD, CH)
    srcS = edge_index[0].reshape(NS, NGF, GF, CH)
    dstS = edge_index[1].reshape(NS, NGF, GF, CH)
    zeros_deg = jnp.zeros((NPAD,), jnp.float32)
    z2d = jnp.zeros((NPAD, DH), jnp.float32)
    keepf = jnp.ones((N,), jnp.float32)
    gw = gate_w[:, 0]

    x_cur = x
    out = None
    for (W, b, p, k) in ((W_in, b_in, p_in, 8000),
                         (W_h1, b_h1, p_h1, 6400),
                         (W_h2, b_h2, p_h2, 5120)):
        deg0, deg1 = _sc_deg(keepf, src3d, dst3d, zeros_deg)
        dis3, hw, hws0, hws1 = _tc_mm_pre(deg0, deg1, x_cur, W)
        a0, a1 = _sc_agg_feat(hws0, hws1, srcS, dstS, z2d)
        h = _tc_finish(a0[:N], a1[:N], hw, dis3, b)
        score = (h @ p) / (jnp.linalg.norm(p) + 1e-16)
        score = jnp.where(keepf > 0, score, -jnp.inf)
        x_cur, keepf, out_l = _tc_pool(h, score, gw, k)
        out = out_l if out is None else out + out_l
    return out
